# Initial kernel scaffold; baseline (speedup 1.0000x reference)
import jax
import jax.numpy as jnp
from jax.experimental import pallas as pl


def kernel(seq):
    raise NotImplementedError("write your pallas kernel here")


# speedup vs baseline: 3.6208x; 3.6208x over previous
import jax
import jax.numpy as jnp
from jax import lax
from jax.experimental import pallas as pl

_H = 14
_ROWS_PER_BLOCK = 2048


def _body(in_ref, out_ref):
    x = in_ref[...]
    idx = x[:, 0:1].astype(jnp.int32)
    col = lax.broadcasted_iota(jnp.int32, (x.shape[0], _H), 1)
    oh = jnp.where((col == idx) & (idx != 0),
                   jnp.asarray(1.0, x.dtype),
                   jnp.asarray(0.0, x.dtype))
    out_ref[...] = jnp.concatenate([oh, x[:, 1:]], axis=-1)


def kernel(seq):
    B, L, F = seq.shape
    rows = B * L
    flat = seq.reshape(rows, F)
    R = _ROWS_PER_BLOCK
    out = pl.pallas_call(
        _body,
        grid=(rows // R,),
        in_specs=[pl.BlockSpec((R, F), lambda i: (i, 0))],
        out_specs=pl.BlockSpec((R, F + _H - 1), lambda i: (i, 0)),
        out_shape=jax.ShapeDtypeStruct((rows, F + _H - 1), seq.dtype),
    )(flat)
    return out.reshape(B, L, F + _H - 1)


# speedup vs baseline: 3.9193x; 1.0824x over previous
import functools

import jax
import jax.numpy as jnp
from jax import lax
from jax.experimental import pallas as pl
from jax.experimental.pallas import tpu as pltpu
from jax.experimental.pallas import tpu_sc as plsc

_H = 14
_F = 64
_FO = _F + _H - 1
_CHUNK = 256


def _row_body(r, in_v, out_v):
    idx = in_v[r, pl.ds(0, 16)][0].astype(jnp.int32)
    lane = lax.broadcasted_iota(jnp.int32, (16,), 0)
    lane = ((lane + 15) % 16) + 1
    oh = jnp.where(lane == idx,
                   jnp.asarray(1.0, jnp.float32),
                   jnp.asarray(0.0, jnp.float32))
    out_v[r, pl.ds(0, 16)] = oh
    out_v[r, pl.ds(14, 16)] = in_v[r, pl.ds(1, 16)]
    out_v[r, pl.ds(30, 16)] = in_v[r, pl.ds(17, 16)]
    out_v[r, pl.ds(46, 16)] = in_v[r, pl.ds(33, 16)]
    out_v[r, pl.ds(61, 16)] = in_v[r, pl.ds(48, 16)]


def _make_sc_kernel(rows):
    info = plsc.get_sparse_core_info()
    nc, ns = info.num_cores, info.num_subcores
    nw = nc * ns
    rows_per_w = rows // nw
    n_chunks = rows_per_w // _CHUNK
    mesh = plsc.VectorSubcoreMesh(core_axis_name="c", subcore_axis_name="s")

    @functools.partial(
        pl.kernel,
        mesh=mesh,
        out_type=jax.ShapeDtypeStruct((rows, _FO), jnp.float32),
        scratch_types=[
            pltpu.VMEM((_CHUNK, _F), jnp.float32),
            pltpu.VMEM((_CHUNK, _FO), jnp.float32),
        ],
    )
    def sc_kernel(in_hbm, out_hbm, in_v, out_v):
        wid = lax.axis_index("s") * nc + lax.axis_index("c")
        w_base = wid * rows_per_w

        def chunk_body(c, _):
            base = w_base + c * _CHUNK
            pltpu.sync_copy(in_hbm.at[pl.ds(base, _CHUNK)], in_v)
            lax.fori_loop(0, _CHUNK, lambda r, _: (_row_body(r, in_v, out_v), 0)[1], 0)
            pltpu.sync_copy(out_v, out_hbm.at[pl.ds(base, _CHUNK)])
            return 0

        lax.fori_loop(0, n_chunks, chunk_body, 0)

    return sc_kernel


def kernel(seq):
    B, L, F = seq.shape
    rows = B * L
    flat = seq.reshape(rows, F)
    out = _make_sc_kernel(rows)(flat)
    return out.reshape(B, L, _FO)
